# baseline (speedup 1.0000x reference)
import jax
import jax.numpy as jnp
from jax.experimental import pallas as pl
from jax.experimental.pallas import tpu as pltpu

TILE = 512
NSPLIT = 4


def _router_kernel(*refs):
    x_refs = refs[:NSPLIT]
    w_ref, bias_ref, out_ref = refs[NSPLIT:]
    q = x_refs[0].shape[1]
    logits = bias_ref[...]
    for k in range(NSPLIT):
        logits = logits + jax.lax.dot_general(
            x_refs[k][...], w_ref[:, k * q:(k + 1) * q],
            dimension_numbers=(((1,), (1,)), ((), ())),
            preferred_element_type=jnp.float32,
        )
    m = jnp.max(logits, axis=-1, keepdims=True)
    e = jnp.exp(logits - m)
    out_ref[...] = e / jnp.sum(e, axis=-1, keepdims=True)


@jax.jit
def kernel(x, W, bias):
    n_tokens, dim = x.shape
    n_experts = W.shape[0]
    q = dim // NSPLIT
    grid = (n_tokens // TILE,)

    def mk(k):
        return pl.BlockSpec((TILE, q), lambda i, k=k: (i, k))

    return pl.pallas_call(
        _router_kernel,
        grid=grid,
        in_specs=[mk(k) for k in range(NSPLIT)]
        + [
            pl.BlockSpec((n_experts, dim), lambda i: (0, 0)),
            pl.BlockSpec((1, n_experts), lambda i: (0, 0)),
        ],
        out_specs=pl.BlockSpec((TILE, n_experts), lambda i: (i, 0)),
        out_shape=jax.ShapeDtypeStruct((n_tokens, n_experts), jnp.float32),
        compiler_params=pltpu.CompilerParams(
            dimension_semantics=("arbitrary",),
        ),
    )(*([x] * NSPLIT), W, bias.reshape(1, n_experts))


# speedup vs baseline: 1.1788x; 1.1788x over previous
import jax
import jax.numpy as jnp
from jax.experimental import pallas as pl
from jax.experimental.pallas import tpu as pltpu

TILE = 1024
NSPLIT = 4


def _router_kernel(*refs):
    x_refs = refs[:NSPLIT]
    w_ref, bias_ref, out_ref = refs[NSPLIT:]
    q = x_refs[0].shape[1]
    logits = bias_ref[...]
    for k in range(NSPLIT):
        logits = logits + jax.lax.dot_general(
            x_refs[k][...], w_ref[:, k * q:(k + 1) * q],
            dimension_numbers=(((1,), (1,)), ((), ())),
            preferred_element_type=jnp.float32,
        )
    m = jnp.max(logits, axis=-1, keepdims=True)
    e = jnp.exp(logits - m)
    out_ref[...] = e / jnp.sum(e, axis=-1, keepdims=True)


@jax.jit
def kernel(x, W, bias):
    n_tokens, dim = x.shape
    n_experts = W.shape[0]
    q = dim // NSPLIT
    grid = (n_tokens // TILE,)

    def mk(k):
        return pl.BlockSpec((TILE, q), lambda i, k=k: (i, k))

    return pl.pallas_call(
        _router_kernel,
        grid=grid,
        in_specs=[mk(k) for k in range(NSPLIT)]
        + [
            pl.BlockSpec((n_experts, dim), lambda i: (0, 0)),
            pl.BlockSpec((1, n_experts), lambda i: (0, 0)),
        ],
        out_specs=pl.BlockSpec((TILE, n_experts), lambda i: (i, 0)),
        out_shape=jax.ShapeDtypeStruct((n_tokens, n_experts), jnp.float32),
        compiler_params=pltpu.CompilerParams(
            dimension_semantics=("parallel",),
        ),
    )(*([x] * NSPLIT), W, bias.reshape(1, n_experts))
